# Initial kernel scaffold; baseline (speedup 1.0000x reference)
import jax
import jax.numpy as jnp
from jax.experimental import pallas as pl


def kernel(x, Wg, W1, b1, W2, b2):
    raise NotImplementedError("write your pallas kernel here")


# speedup vs baseline: 1.0488x; 1.0488x over previous
import functools

import jax
import jax.numpy as jnp
from jax import lax
from jax.experimental import pallas as pl
from jax.experimental.pallas import tpu as pltpu
from jax.experimental.pallas import tpu_sc as plsc

N = 2048
D = 1024
E = 64
K = 2
H = 2048
CAP = (K * N) // E

NC = 2
NS = 16
NW = NC * NS
LANES = 16

HT = 2
HTILE = H // HT


def _routing_body(x_ref, wg_ref, tok_ref, wslot_ref, flat0_ref, flat1_ref,
                  laux_ref):
    x = x_ref[...]
    logits = jnp.dot(x, wg_ref[...], preferred_element_type=jnp.float32)
    mx = jnp.max(logits, axis=-1, keepdims=True)
    ex = jnp.exp(logits - mx)
    gates = ex / jnp.sum(ex, axis=-1, keepdims=True)

    iota_e = lax.broadcasted_iota(jnp.int32, (N, E), 1)
    v0 = jnp.max(gates, axis=-1, keepdims=True)
    i0 = jnp.min(jnp.where(gates == v0, iota_e, E), axis=-1, keepdims=True)
    m0 = (iota_e == i0).astype(jnp.float32)
    g1 = jnp.where(m0 > 0, -jnp.inf, gates)
    v1 = jnp.max(g1, axis=-1, keepdims=True)
    i1 = jnp.min(jnp.where(g1 == v1, iota_e, E), axis=-1, keepdims=True)
    m1 = (iota_e == i1).astype(jnp.float32)
    denom = v0 + v1 + 1e-9
    w0 = v0 / denom
    w1 = v1 / denom

    me = jnp.mean(gates, axis=0, keepdims=True)
    ce = jnp.mean(m0, axis=0, keepdims=True)
    laux_ref[...] = jnp.sum(me * ce, keepdims=True) * float(E)

    ri = lax.broadcasted_iota(jnp.int32, (N, N), 0)
    ci = lax.broadcasted_iota(jnp.int32, (N, N), 1)
    T = (ri >= ci).astype(jnp.bfloat16)
    m0b = m0.astype(jnp.bfloat16)
    m1b = m1.astype(jnp.bfloat16)
    loc0 = jnp.dot(T, m0b, preferred_element_type=jnp.float32) - 1.0
    c0 = jnp.sum(m0, axis=0, keepdims=True)
    loc1 = jnp.dot(T, m1b, preferred_element_type=jnp.float32) - 1.0 + c0
    pos0 = jnp.sum(loc0 * m0, axis=-1, keepdims=True)
    pos1 = jnp.sum(loc1 * m1, axis=-1, keepdims=True)
    keep0 = (pos0 < CAP).astype(jnp.float32)
    keep1 = (pos1 < CAP).astype(jnp.float32)
    w0f = w0 * keep0
    w1f = w1 * keep1

    iota_c = lax.broadcasted_iota(jnp.int32, (N, CAP), 1).astype(jnp.float32)
    ohc0 = (iota_c == pos0).astype(jnp.float32)
    ohc1 = (iota_c == pos1).astype(jnp.float32)
    nvec = lax.broadcasted_iota(jnp.int32, (N, 1), 0).astype(jnp.float32)

    def dotT(a, b):
        return lax.dot_general(a, b, (((0,), (0,)), ((), ())),
                               preferred_element_type=jnp.float32)

    filled = dotT(m0, ohc0) + dotT(m1, ohc1)
    tokf = dotT(m0 * nvec, ohc0) + dotT(m1 * nvec, ohc1)
    wslot = dotT(m0 * w0f, ohc0) + dotT(m1 * w1f, ohc1)

    ie = lax.broadcasted_iota(jnp.int32, (E, CAP), 0)
    ic = lax.broadcasted_iota(jnp.int32, (E, CAP), 1)
    fi = (ie * CAP + ic).astype(jnp.float32)
    zcand = jnp.where(filled == 0, fi, float(E * CAP))
    zidx = jnp.minimum(jnp.min(zcand), float(E * CAP - 1))

    flat0 = jnp.where(keep0 > 0, i0.astype(jnp.float32) * CAP + pos0, zidx)
    flat1 = jnp.where(keep1 > 0, i1.astype(jnp.float32) * CAP + pos1, zidx)

    tok_ref[...] = tokf.astype(jnp.int32)
    wslot_ref[...] = wslot
    flat0_ref[...] = flat0.astype(jnp.int32)
    flat1_ref[...] = flat1.astype(jnp.int32)


def _routing(x, Wg):
    return pl.pallas_call(
        _routing_body,
        out_shape=(
            jax.ShapeDtypeStruct((E, CAP), jnp.int32),
            jax.ShapeDtypeStruct((E, CAP), jnp.float32),
            jax.ShapeDtypeStruct((N, 1), jnp.int32),
            jax.ShapeDtypeStruct((N, 1), jnp.int32),
            jax.ShapeDtypeStruct((1, 1), jnp.float32),
        ),
    )(x, Wg)


def _ffn_body(disp_ref, w1_ref, b1_ref, w2_ref, b2_ref, wslot_ref, out_ref):
    e = pl.program_id(0)
    t = pl.program_id(1)
    xb = disp_ref[...]
    h = jnp.maximum(
        jnp.dot(xb, w1_ref[0], preferred_element_type=jnp.float32)
        + b1_ref[0], 0.0)
    part = jnp.dot(h, w2_ref[0], preferred_element_type=jnp.float32)

    @pl.when(t == 0)
    def _init():
        out_ref[...] = part + b2_ref[0]

    @pl.when(t > 0)
    def _acc():
        out_ref[...] = out_ref[...] + part

    @pl.when(t == HT - 1)
    def _scale():
        wrow = wslot_ref[pl.ds(e, 1), :]
        eye = (lax.broadcasted_iota(jnp.int32, (CAP, CAP), 0)
               == lax.broadcasted_iota(jnp.int32, (CAP, CAP), 1))
        diag = jnp.where(eye, jnp.broadcast_to(wrow, (CAP, CAP)), 0.0)
        out_ref[...] = jnp.dot(diag, out_ref[...],
                               preferred_element_type=jnp.float32)


def _ffn(disp, W1, b1, W2, b2, wslot):
    b1r = b1.reshape(E, 1, H)
    b2r = b2.reshape(E, 1, D)
    return pl.pallas_call(
        _ffn_body,
        grid=(E, HT),
        in_specs=[
            pl.BlockSpec((CAP, D), lambda e, t: (e, 0)),
            pl.BlockSpec((1, D, HTILE), lambda e, t: (e, 0, t)),
            pl.BlockSpec((1, 1, HTILE), lambda e, t: (e, 0, t)),
            pl.BlockSpec((1, HTILE, D), lambda e, t: (e, t, 0)),
            pl.BlockSpec((1, 1, D), lambda e, t: (e, 0, 0)),
            pl.BlockSpec((E, CAP), lambda e, t: (0, 0)),
        ],
        out_specs=pl.BlockSpec((CAP, D), lambda e, t: (e, 0)),
        out_shape=jax.ShapeDtypeStruct((E * CAP, D), jnp.float32),
    )(disp, W1, b1r, W2, b2r, wslot)


_DISP_RPW = (E * CAP) // NW
_DISP_CH = 64


@functools.lru_cache(maxsize=None)
def _make_dispatch_sc():
    mesh = plsc.VectorSubcoreMesh(core_axis_name="c", subcore_axis_name="s")

    @functools.partial(
        pl.kernel, mesh=mesh,
        out_type=jax.ShapeDtypeStruct((E * CAP, D), jnp.float32),
        scratch_types=[
            pltpu.VMEM((_DISP_CH,), jnp.int32),
            pltpu.VMEM((_DISP_CH, D), jnp.float32),
            pltpu.SemaphoreType.DMA,
        ],
    )
    def _dispatch_sc(x_hbm, tok_hbm, out_hbm, idx_v, rows_v, sem):
        wid = lax.axis_index("s") * NC + lax.axis_index("c")
        base = wid * _DISP_RPW
        for j in range(_DISP_RPW // _DISP_CH):
            off = base + j * _DISP_CH
            pltpu.sync_copy(tok_hbm.at[pl.ds(off, _DISP_CH)], idx_v)
            pltpu.async_copy(x_hbm.at[idx_v], rows_v, sem).wait()
            pltpu.sync_copy(rows_v, out_hbm.at[pl.ds(off, _DISP_CH)])

    return _dispatch_sc


_CMB_TPW = N // NW
_CMB_CH = 32


@functools.lru_cache(maxsize=None)
def _make_combine_sc():
    mesh = plsc.VectorSubcoreMesh(core_axis_name="c", subcore_axis_name="s")

    @functools.partial(
        pl.kernel, mesh=mesh,
        out_type=jax.ShapeDtypeStruct((N, D), jnp.float32),
        scratch_types=[
            pltpu.VMEM((_CMB_CH,), jnp.int32),
            pltpu.VMEM((_CMB_CH,), jnp.int32),
            pltpu.VMEM((_CMB_CH, D), jnp.float32),
            pltpu.VMEM((_CMB_CH, D), jnp.float32),
            pltpu.SemaphoreType.DMA,
        ],
    )
    def _combine_sc(eoutw_hbm, flat0_hbm, flat1_hbm, y_hbm,
                    idx0_v, idx1_v, b0_v, b1_v, sem):
        wid = lax.axis_index("s") * NC + lax.axis_index("c")
        base = wid * _CMB_TPW
        for j in range(_CMB_TPW // _CMB_CH):
            off = base + j * _CMB_CH
            pltpu.sync_copy(flat0_hbm.at[pl.ds(off, _CMB_CH)], idx0_v)
            pltpu.sync_copy(flat1_hbm.at[pl.ds(off, _CMB_CH)], idx1_v)
            pltpu.async_copy(eoutw_hbm.at[idx0_v], b0_v, sem).wait()
            pltpu.async_copy(eoutw_hbm.at[idx1_v], b1_v, sem).wait()

            def add_row(r, _):
                for c in range(D // LANES):
                    sl = pl.ds(c * LANES, LANES)
                    b0_v[r, sl] = b0_v[r, sl] + b1_v[r, sl]
                return 0

            lax.fori_loop(0, _CMB_CH, add_row, 0)
            pltpu.sync_copy(b0_v, y_hbm.at[pl.ds(off, _CMB_CH)])

    return _combine_sc


def kernel(x, Wg, W1, b1, W2, b2):
    tok, wslot, flat0, flat1, laux = _routing(x, Wg)
    disp = _make_dispatch_sc()(x, tok.reshape(E * CAP))
    eoutw = _ffn(disp, W1, b1, W2, b2, wslot)
    y = _make_combine_sc()(eoutw, flat0.reshape(N), flat1.reshape(N))
    return y, laux.reshape(())
